# baseline (speedup 1.0000x reference)
import functools
import math

import jax
import jax.numpy as jnp
from jax import lax
from jax.experimental import pallas as pl
from jax.experimental.pallas import tpu as pltpu
from jax.experimental.pallas import tpu_sc as plsc

D_MODEL = 64
SEQ = 200
BATCH = 1024
N_VOCAB = 1000000
CHUNK = 128
CPS = BATCH // CHUNK
N_CHUNKS = SEQ * CPS
NC, NS = 2, 16
NW = NC * NS
CPW = N_CHUNKS // NW
SCALE = math.sqrt(D_MODEL)
LANES = 16
NGROUPS = CHUNK // LANES
SLAB = 16


def _body(ih_hbm, pv_hbm, tbl_hbm, pe_hbm, out_hbm,
          ih_v, pv_v, pe_v, g0, g1, o0, o1, gs0, gs1, os0, os1):
    w = lax.axis_index("s") * NC + lax.axis_index("c")
    c0 = w * CPW
    s_start = c0 // CPS
    sbase = jnp.minimum((s_start // 8) * 8, SEQ - SLAB)

    pltpu.sync_copy(ih_hbm.at[pl.ds(sbase, SLAB)], ih_v)
    pltpu.sync_copy(pv_hbm.at[pl.ds(sbase, SLAB)], pv_v)
    pltpu.sync_copy(pe_hbm.at[pl.ds(sbase, SLAB)], pe_v)

    bufs = ((g0, o0, gs0, os0), (g1, o1, gs1, os1))
    row_vecs = [lax.iota(jnp.int32, LANES) + gi * LANES
                for gi in range(NGROUPS)]

    def chunk_pos(c):
        return c // CPS, c % CPS

    def gather_src(c):
        s, bb = chunk_pos(c)
        return tbl_hbm.at[ih_v.at[s - sbase, pl.ds(bb * CHUNK, CHUNK)]]

    def out_dst(c):
        s, bb = chunk_pos(c)
        return out_hbm.at[s, :, pl.ds(bb * CHUNK, CHUNK)]

    for p in range(2):
        pltpu.async_copy(gather_src(c0 + p), bufs[p][0], bufs[p][2])

    def round_body(r, carry):
        for p in range(2):
            g, o, gsem, osem = bufs[p]
            k = 2 * r + p
            c = c0 + k
            s, bb = chunk_pos(c)
            s_rel = s - sbase
            pltpu.make_async_copy(gather_src(c), g, gsem).wait()

            @pl.when(r >= 1)
            def _wait_prev_out():
                pltpu.make_async_copy(o, out_dst(c - 2), osem).wait()

            par_vecs = [pv_v[s_rel, pl.ds(bb * CHUNK + gi * LANES, LANES)]
                        for gi in range(NGROUPS)]

            def d_body(d, _):
                pe_sd = pe_v[s_rel, pl.ds(d * LANES, LANES)]
                for gi in range(NGROUPS):
                    vals = plsc.load_gather(
                        g, [row_vecs[gi], par_vecs[gi] + d])
                    o[d, pl.ds(gi * LANES, LANES)] = vals * SCALE + pe_sd
                return 0

            lax.fori_loop(0, D_MODEL, d_body, 0)
            pltpu.async_copy(o, out_dst(c), osem)

            @pl.when(k + 2 < CPW)
            def _issue_next_gather():
                pltpu.async_copy(gather_src(c + 2), g, gsem)
        return carry

    lax.fori_loop(0, CPW // 2, round_body, 0)

    for p in range(2):
        c = c0 + CPW - 2 + p
        pltpu.make_async_copy(bufs[p][1], out_dst(c), bufs[p][3]).wait()


@jax.jit
def _emb_pe(idx_hi, par64, tbl2, pe_flat):
    mesh = plsc.VectorSubcoreMesh(core_axis_name="c", subcore_axis_name="s")
    return pl.kernel(
        _body,
        out_type=jax.ShapeDtypeStruct((SEQ, D_MODEL, BATCH), jnp.float32),
        mesh=mesh,
        compiler_params=pltpu.CompilerParams(
            use_tc_tiling_on_sc=True, needs_layout_passes=False),
        scratch_types=[
            pltpu.VMEM((SLAB, BATCH), jnp.int32),
            pltpu.VMEM((SLAB, BATCH), jnp.int32),
            pltpu.VMEM((SLAB, BATCH), jnp.float32),
            pltpu.VMEM((CHUNK, 2 * D_MODEL), jnp.float32),
            pltpu.VMEM((CHUNK, 2 * D_MODEL), jnp.float32),
            pltpu.VMEM((D_MODEL, CHUNK), jnp.float32),
            pltpu.VMEM((D_MODEL, CHUNK), jnp.float32),
            pltpu.SemaphoreType.DMA,
            pltpu.SemaphoreType.DMA,
            pltpu.SemaphoreType.DMA,
            pltpu.SemaphoreType.DMA,
        ],
    )(idx_hi, par64, tbl2, pe_flat)


def kernel(sparse_input, table, pe):
    seq, batch = sparse_input.shape
    idx = sparse_input.astype(jnp.int32)
    idx_hi = idx >> 1
    par64 = (idx & 1) * D_MODEL
    tbl2 = table.reshape(N_VOCAB // 2, 2 * D_MODEL)
    pe2 = pe[:seq, 0, :]
    pe_b = jnp.broadcast_to(pe2[:, :, None],
                            (seq, D_MODEL, LANES)).reshape(seq, batch)
    out_sdb = _emb_pe(idx_hi, par64, tbl2, pe_b)
    return jnp.transpose(out_sdb, (0, 2, 1))


# speedup vs baseline: 1.2717x; 1.2717x over previous
import functools
import math

import jax
import jax.numpy as jnp
from jax import lax
from jax.experimental import pallas as pl
from jax.experimental.pallas import tpu as pltpu
from jax.experimental.pallas import tpu_sc as plsc

D_MODEL = 64
SEQ = 200
BATCH = 1024
N_ROWS = SEQ * BATCH
CHUNK = 128
CHUNKS_PER_SEQ = BATCH // CHUNK
N_CHUNKS = N_ROWS // CHUNK
NC, NS = 2, 16
NW = NC * NS
CPW = N_CHUNKS // NW
SCALE = math.sqrt(D_MODEL)
LANES = 16
GROUPS = D_MODEL // LANES
ROW_UNROLL = 4


def _body(idx_hbm, table_hbm, pe_hbm, out_hbm,
          idx_all, a0, a1, o0, o1, pe_v, g0, g1, s0, s1):
    w = lax.axis_index("s") * NC + lax.axis_index("c")
    c0 = w * CPW
    pltpu.sync_copy(pe_hbm, pe_v)
    pltpu.sync_copy(idx_hbm.at[w], idx_all)
    bufs = ((a0, o0, g0, s0), (a1, o1, g1, s1))

    for p in range(2):
        pltpu.async_copy(table_hbm.at[idx_all.at[p]], bufs[p][0], bufs[p][2])

    def round_body(r, carry):
        for p in range(2):
            a, o, gsem, osem = bufs[p]
            k = 2 * r + p
            c = c0 + k
            pltpu.make_async_copy(table_hbm.at[idx_all.at[k]], a, gsem).wait()

            @pl.when(r >= 1)
            def _wait_prev_scatter():
                pltpu.make_async_copy(
                    o, out_hbm.at[pl.ds((c - 2) * CHUNK, CHUNK)], osem).wait()

            s_off = (c // CHUNKS_PER_SEQ) * D_MODEL
            pe_vecs = [pe_v[pl.ds(s_off + LANES * j, LANES)]
                       for j in range(GROUPS)]

            def row_body(i, _):
                for u in range(ROW_UNROLL):
                    row = i * ROW_UNROLL + u
                    for j in range(GROUPS):
                        o[row, pl.ds(LANES * j, LANES)] = (
                            a[row, pl.ds(LANES * j, LANES)] * SCALE
                            + pe_vecs[j])
                return 0

            lax.fori_loop(0, CHUNK // ROW_UNROLL, row_body, 0)
            pltpu.async_copy(o, out_hbm.at[pl.ds(c * CHUNK, CHUNK)], osem)

            @pl.when(k + 2 < CPW)
            def _issue_next_gather():
                pltpu.async_copy(
                    table_hbm.at[idx_all.at[k + 2]], a, gsem)
        return carry

    lax.fori_loop(0, CPW // 2, round_body, 0)

    for p in range(2):
        c = c0 + CPW - 2 + p
        pltpu.make_async_copy(
            bufs[p][1], out_hbm.at[pl.ds(c * CHUNK, CHUNK)], bufs[p][3]).wait()


@jax.jit
def _emb_pe(idx2d, table, pe_flat):
    mesh = plsc.VectorSubcoreMesh(core_axis_name="c", subcore_axis_name="s")
    return pl.kernel(
        _body,
        out_type=jax.ShapeDtypeStruct((N_ROWS, D_MODEL), jnp.float32),
        mesh=mesh,
        compiler_params=pltpu.CompilerParams(use_tc_tiling_on_sc=False),
        scratch_types=[
            pltpu.VMEM((CPW, CHUNK), jnp.int32),
            pltpu.VMEM((CHUNK, D_MODEL), jnp.float32),
            pltpu.VMEM((CHUNK, D_MODEL), jnp.float32),
            pltpu.VMEM((CHUNK, D_MODEL), jnp.float32),
            pltpu.VMEM((CHUNK, D_MODEL), jnp.float32),
            pltpu.VMEM((SEQ * D_MODEL,), jnp.float32),
            pltpu.SemaphoreType.DMA,
            pltpu.SemaphoreType.DMA,
            pltpu.SemaphoreType.DMA,
            pltpu.SemaphoreType.DMA,
        ],
    )(idx2d, table, pe_flat)


def kernel(sparse_input, table, pe):
    seq, batch = sparse_input.shape
    idx2d = sparse_input.astype(jnp.int32).reshape(NW, CPW, CHUNK)
    pe_flat = pe[:seq].reshape(seq * D_MODEL)
    out = _emb_pe(idx2d, table, pe_flat)
    return out.reshape(seq, batch, D_MODEL)


# speedup vs baseline: 1.2745x; 1.0023x over previous
import math

import jax
import jax.numpy as jnp
from jax import lax
from jax.experimental import pallas as pl
from jax.experimental.pallas import tpu as pltpu
from jax.experimental.pallas import tpu_sc as plsc

D_MODEL = 64
SEQ = 200
BATCH = 1024
N_ROWS = SEQ * BATCH
CHUNK = 128
CHUNKS_PER_SEQ = BATCH // CHUNK
N_CHUNKS = N_ROWS // CHUNK
NC, NS = 2, 16
NW = NC * NS
CPW = N_CHUNKS // NW
SCALE = math.sqrt(D_MODEL)
LANES = 16
GROUPS = D_MODEL // LANES
ROW_UNROLL = 4


def _body(idx_hbm, table_hbm, pe_hbm, out_hbm,
          idx_all, a0, a1, o0, o1, pe_v, g0, g1, s0, s1):
    w = lax.axis_index("s") * NC + lax.axis_index("c")
    c0 = w * CPW
    pltpu.sync_copy(pe_hbm, pe_v)
    pltpu.sync_copy(idx_hbm.at[w], idx_all)
    bufs = ((a0, o0, g0, s0), (a1, o1, g1, s1))

    for p in range(2):
        pltpu.async_copy(table_hbm.at[idx_all.at[p]], bufs[p][0], bufs[p][2])

    def round_body(r, carry):
        for p in range(2):
            a, o, gsem, osem = bufs[p]
            k = 2 * r + p
            c = c0 + k
            pltpu.make_async_copy(table_hbm.at[idx_all.at[k]], a, gsem).wait()

            @pl.when(r >= 1)
            def _wait_prev_scatter():
                pltpu.make_async_copy(
                    o, out_hbm.at[pl.ds((c - 2) * CHUNK, CHUNK)], osem).wait()

            s_off = (c // CHUNKS_PER_SEQ) * D_MODEL
            pe_vecs = [pe_v[pl.ds(s_off + LANES * j, LANES)]
                       for j in range(GROUPS)]

            def row_body(i, _):
                for u in range(ROW_UNROLL):
                    row = i * ROW_UNROLL + u
                    for j in range(GROUPS):
                        o[row, pl.ds(LANES * j, LANES)] = (
                            a[row, pl.ds(LANES * j, LANES)] * SCALE
                            + pe_vecs[j])
                return 0

            lax.fori_loop(0, CHUNK // ROW_UNROLL, row_body, 0)
            pltpu.async_copy(o, out_hbm.at[pl.ds(c * CHUNK, CHUNK)], osem)

            @pl.when(k + 2 < CPW)
            def _issue_next_gather():
                pltpu.async_copy(
                    table_hbm.at[idx_all.at[k + 2]], a, gsem)
        return carry

    lax.fori_loop(0, CPW // 2, round_body, 0)

    for p in range(2):
        c = c0 + CPW - 2 + p
        pltpu.make_async_copy(
            bufs[p][1], out_hbm.at[pl.ds(c * CHUNK, CHUNK)], bufs[p][3]).wait()


@jax.jit
def _emb_pe(idx2d, table, pe_flat):
    mesh = plsc.VectorSubcoreMesh(core_axis_name="c", subcore_axis_name="s")
    return pl.kernel(
        _body,
        out_type=jax.ShapeDtypeStruct((N_ROWS, D_MODEL), jnp.float32),
        mesh=mesh,
        compiler_params=pltpu.CompilerParams(use_tc_tiling_on_sc=False),
        scratch_types=[
            pltpu.VMEM((CPW, CHUNK), jnp.int32),
            pltpu.VMEM((CHUNK, D_MODEL), jnp.float32),
            pltpu.VMEM((CHUNK, D_MODEL), jnp.float32),
            pltpu.VMEM((CHUNK, D_MODEL), jnp.float32),
            pltpu.VMEM((CHUNK, D_MODEL), jnp.float32),
            pltpu.VMEM((SEQ * D_MODEL,), jnp.float32),
            pltpu.SemaphoreType.DMA,
            pltpu.SemaphoreType.DMA,
            pltpu.SemaphoreType.DMA,
            pltpu.SemaphoreType.DMA,
        ],
    )(idx2d, table, pe_flat)


def kernel(sparse_input, table, pe):
    seq, batch = sparse_input.shape
    idx2d = sparse_input.astype(jnp.int32).reshape(NW, CPW, CHUNK)
    pe_flat = pe[:seq].reshape(seq * D_MODEL)
    out = _emb_pe(idx2d, table, pe_flat)
    return out.reshape(seq, batch, D_MODEL)
